# baseline (speedup 1.0000x reference)
import functools

import jax
import jax.numpy as jnp
from jax import lax
from jax.experimental import pallas as pl
from jax.experimental.pallas import tpu as pltpu
from jax.experimental.pallas import tpu_sc as plsc

NC = 2
NS = 16
NW = NC * NS
CH = 80


def _sc_aggregate(x, src, dst, n_acc, e_per_w):
    n, d = x.shape
    stripe = n_acc // NS
    chunks = e_per_w // CH
    pairs = chunks // 2
    odd_tail = chunks % 2 == 1

    mesh = plsc.VectorSubcoreMesh(core_axis_name="c", subcore_axis_name="s")

    @functools.partial(
        pl.kernel,
        out_type=jax.ShapeDtypeStruct((NC, n_acc, d), jnp.float32),
        mesh=mesh,
        scratch_types=[
            pltpu.VMEM((e_per_w,), jnp.int32),
            pltpu.VMEM((e_per_w,), jnp.int32),
            pltpu.VMEM((CH, d), jnp.float32),
            pltpu.VMEM((CH, d), jnp.float32),
            pltpu.VMEM_SHARED((n_acc, d), jnp.float32),
            pltpu.SemaphoreType.DMA,
            pltpu.SemaphoreType.DMA,
        ],
    )
    def k(x_hbm, src_hbm, dst_hbm, out_hbm, src_v, dst_v, rows0, rows1, acc, sem0, sem1):
        c = lax.axis_index("c")
        s = lax.axis_index("s")
        wid = s * NC + c

        pltpu.sync_copy(src_hbm.at[pl.ds(wid * e_per_w, e_per_w)], src_v)
        pltpu.sync_copy(dst_hbm.at[pl.ds(wid * e_per_w, e_per_w)], dst_v)
        pltpu.async_copy(x_hbm.at[src_v.at[pl.ds(0, CH)]], rows0, sem0)

        zvec = jnp.zeros((16,), jnp.float32)

        def zrow(r, _):
            for cc in range(d // 16):
                rows1[r, pl.ds(cc * 16, 16)] = zvec
            return 0

        lax.fori_loop(0, CH, zrow, 0)

        zfull = stripe // CH
        ztail = stripe - zfull * CH

        def zcopy(r, _):
            pltpu.sync_copy(rows1, acc.at[pl.ds(s * stripe + r * CH, CH), :])
            return 0

        lax.fori_loop(0, zfull, zcopy, 0)
        if ztail:
            pltpu.sync_copy(
                rows1.at[pl.ds(0, ztail), :],
                acc.at[pl.ds(s * stripe + zfull * CH, ztail), :],
            )
        plsc.subcore_barrier()

        def pair(g, _):
            j0 = 2 * g
            pltpu.async_copy(x_hbm.at[src_v.at[pl.ds((j0 + 1) * CH, CH)]], rows1, sem1)
            pltpu.make_async_copy(x_hbm.at[src_v.at[pl.ds(j0 * CH, CH)]], rows0, sem0).wait()
            pltpu.sync_copy(rows0, acc.at[dst_v.at[pl.ds(j0 * CH, CH)]], add=True)

            @pl.when(g + 1 < pairs)
            def _():
                pltpu.async_copy(x_hbm.at[src_v.at[pl.ds((j0 + 2) * CH, CH)]], rows0, sem0)

            pltpu.make_async_copy(x_hbm.at[src_v.at[pl.ds((j0 + 1) * CH, CH)]], rows1, sem1).wait()
            pltpu.sync_copy(rows1, acc.at[dst_v.at[pl.ds((j0 + 1) * CH, CH)]], add=True)
            return 0

        lax.fori_loop(0, pairs, pair, 0)
        if odd_tail:
            jt = chunks - 1
            pltpu.async_copy(x_hbm.at[src_v.at[pl.ds(jt * CH, CH)]], rows0, sem0).wait()
            pltpu.sync_copy(rows0, acc.at[dst_v.at[pl.ds(jt * CH, CH)]], add=True)
        plsc.subcore_barrier()

        pltpu.sync_copy(
            acc.at[pl.ds(s * stripe, stripe), :],
            out_hbm.at[c, pl.ds(s * stripe, stripe), :],
        )

    return k(x, src, dst)


def _tc_combine_matmul(partials, W, b, n, block_rows):
    d_in = partials.shape[-1]
    d_out = W.shape[0]
    grid = (n // block_rows,)

    def body(p_ref, w_ref, b_ref, o_ref):
        ssum = p_ref[0] + p_ref[1]
        o_ref[...] = (
            lax.dot_general(
                ssum, w_ref[...], (((1,), (1,)), ((), ())),
                preferred_element_type=jnp.float32,
            )
            + b_ref[...]
        )

    return pl.pallas_call(
        body,
        grid=grid,
        in_specs=[
            pl.BlockSpec((NC, block_rows, d_in), lambda i: (0, i, 0)),
            pl.BlockSpec((d_out, d_in), lambda i: (0, 0)),
            pl.BlockSpec((1, d_out), lambda i: (0, 0)),
        ],
        out_specs=pl.BlockSpec((block_rows, d_out), lambda i: (i, 0)),
        out_shape=jax.ShapeDtypeStruct((n, d_out), jnp.float32),
    )(partials, W, b.reshape(1, d_out))


def kernel(x, edge_index, W, b):
    n, d = x.shape
    e = edge_index.shape[1]

    assert e % (NW * CH) == 0
    e_per_w = e // NW

    n_acc = -(-n // (8 * NS)) * (8 * NS)

    src = edge_index[0]
    dst = edge_index[1]

    partials = _sc_aggregate(x, src, dst, n_acc, e_per_w)

    block_rows = 2000 if n % 2000 == 0 else (400 if n % 400 == 0 else 16)
    return _tc_combine_matmul(partials, W, b, n, block_rows)


# speedup vs baseline: 1.1208x; 1.1208x over previous
import functools

import jax
import jax.numpy as jnp
from jax import lax
from jax.experimental import pallas as pl
from jax.experimental.pallas import tpu as pltpu
from jax.experimental.pallas import tpu_sc as plsc

NC = 2
NS = 16
NW = NC * NS
CH = 128


def _sc_aggregate(x, src_m, dst_m, n_acc, chunks_per_w):
    n, d = x.shape
    stripe = n_acc // NS
    assert stripe % CH == 0
    assert chunks_per_w % 4 == 0
    half = chunks_per_w // 2
    pairs = half // 2

    mesh = plsc.VectorSubcoreMesh(core_axis_name="c", subcore_axis_name="s")

    @functools.partial(
        pl.kernel,
        out_type=jax.ShapeDtypeStruct((NC, n_acc, d), jnp.float32),
        mesh=mesh,
        scratch_types=[
            pltpu.VMEM((half, CH), jnp.int32),
            pltpu.VMEM((half, CH), jnp.int32),
            pltpu.VMEM((CH, d), jnp.float32),
            pltpu.VMEM((CH, d), jnp.float32),
            pltpu.VMEM_SHARED((n_acc, d), jnp.float32),
            pltpu.SemaphoreType.DMA,
            pltpu.SemaphoreType.DMA,
        ],
    )
    def k(x_hbm, srcm_hbm, dstm_hbm, out_hbm, src_v, dst_v, rows0, rows1, acc, sem0, sem1):
        c = lax.axis_index("c")
        s = lax.axis_index("s")
        wid = s * NC + c

        def stage(h):
            base = wid * chunks_per_w + h * half
            pltpu.sync_copy(srcm_hbm.at[pl.ds(base, half), :], src_v)
            pltpu.sync_copy(dstm_hbm.at[pl.ds(base, half), :], dst_v)

        with jax.named_scope("stage_idx"):
            stage(0)
        pltpu.async_copy(x_hbm.at[src_v.at[0]], rows0, sem0)

        zvec = jnp.zeros((16,), jnp.float32)

        with jax.named_scope("zero_acc"):
            def zrow(r, _):
                for cc in range(d // 16):
                    rows1[r, pl.ds(cc * 16, 16)] = zvec
                return 0

            lax.fori_loop(0, CH, zrow, 0)

            def zcopy(r, _):
                pltpu.sync_copy(rows1, acc.at[pl.ds(s * stripe + r * CH, CH), :])
                return 0

            lax.fori_loop(0, stripe // CH, zcopy, 0)
            plsc.subcore_barrier()

        for h in range(2):
            if h:
                with jax.named_scope("stage_idx"):
                    stage(h)
                pltpu.async_copy(x_hbm.at[src_v.at[0]], rows0, sem0)

            def pair(g, _):
                j0 = 2 * g
                pltpu.async_copy(x_hbm.at[src_v.at[j0 + 1]], rows1, sem1)
                pltpu.make_async_copy(x_hbm.at[src_v.at[j0]], rows0, sem0).wait()
                pltpu.sync_copy(rows0, acc.at[dst_v.at[j0]], add=True)

                @pl.when(g + 1 < pairs)
                def _():
                    pltpu.async_copy(x_hbm.at[src_v.at[j0 + 2]], rows0, sem0)

                pltpu.make_async_copy(x_hbm.at[src_v.at[j0 + 1]], rows1, sem1).wait()
                pltpu.sync_copy(rows1, acc.at[dst_v.at[j0 + 1]], add=True)
                return 0

            with jax.named_scope("edge_loop"):
                lax.fori_loop(0, pairs, pair, 0)
        plsc.subcore_barrier()

        with jax.named_scope("writeout"):
            pltpu.sync_copy(
                acc.at[pl.ds(s * stripe, stripe), :],
                out_hbm.at[c, pl.ds(s * stripe, stripe), :],
            )

    return k(x, src_m, dst_m)


def _tc_pad_indices(edge_index, n, rows_pad):
    e = edge_index.shape[1]
    rows_in = e // CH
    br = 320
    blocks = rows_pad // br
    max_in_block = -(-e // (br * CH)) - 1

    def body(e_ref, s_ref, d_ref):
        i = pl.program_id(0)
        rows = jax.lax.broadcasted_iota(jnp.int32, (br, CH), 0) + i * br
        lane = jax.lax.broadcasted_iota(jnp.int32, (br, CH), 1)
        valid = rows < rows_in
        pad_src = (rows * CH + lane) % n
        pad_dst = n + lane
        esrc = e_ref[0].reshape(br, CH)
        edst = e_ref[1].reshape(br, CH)
        s_ref[...] = jnp.where(valid, esrc, pad_src)
        d_ref[...] = jnp.where(valid, edst, pad_dst)

    return pl.pallas_call(
        body,
        grid=(blocks,),
        in_specs=[
            pl.BlockSpec((2, br * CH), lambda i: (0, jnp.minimum(i, max_in_block)))
        ],
        out_specs=[
            pl.BlockSpec((br, CH), lambda i: (i, 0)),
            pl.BlockSpec((br, CH), lambda i: (i, 0)),
        ],
        out_shape=[
            jax.ShapeDtypeStruct((rows_pad, CH), jnp.int32),
            jax.ShapeDtypeStruct((rows_pad, CH), jnp.int32),
        ],
    )(edge_index)


def _tc_combine_matmul(partials, W, b, n, block_rows):
    d_in = partials.shape[-1]
    d_out = W.shape[0]
    grid = (n // block_rows,)

    def body(p_ref, w_ref, b_ref, o_ref):
        ssum = p_ref[0] + p_ref[1]
        o_ref[...] = (
            lax.dot_general(
                ssum, w_ref[...], (((1,), (1,)), ((), ())),
                preferred_element_type=jnp.float32,
            )
            + b_ref[...]
        )

    return pl.pallas_call(
        body,
        grid=grid,
        in_specs=[
            pl.BlockSpec((NC, block_rows, d_in), lambda i: (0, i, 0)),
            pl.BlockSpec((d_out, d_in), lambda i: (0, 0)),
            pl.BlockSpec((1, d_out), lambda i: (0, 0)),
        ],
        out_specs=pl.BlockSpec((block_rows, d_out), lambda i: (i, 0)),
        out_shape=jax.ShapeDtypeStruct((n, d_out), jnp.float32),
    )(partials, W, b.reshape(1, d_out))


def kernel(x, edge_index, W, b):
    n, d = x.shape
    e = edge_index.shape[1]

    chunks_per_w = -(-e // (NW * CH * 8)) * 8
    e_pad = NW * CH * chunks_per_w

    n_acc = -(-(n + CH) // (16 * NS)) * (16 * NS)

    src_m, dst_m = _tc_pad_indices(edge_index, n, e_pad // CH)

    partials = _sc_aggregate(x, src_m, dst_m, n_acc, chunks_per_w)

    block_rows = 2000 if n % 2000 == 0 else (400 if n % 400 == 0 else 16)
    return _tc_combine_matmul(partials, W, b, n, block_rows)
